# baseline (speedup 1.0000x reference)
import jax
import jax.numpy as jnp
from jax.experimental import pallas as pl
from jax.experimental.pallas import tpu as pltpu
from jax.experimental.pallas import tpu_sc as plsc

_WINDOW = 128


def kernel(txt_inds, table):
    batch, seq = txt_inds.shape
    vocab, dim = table.shape
    n = batch * seq
    idx = txt_inds.reshape(1, n).astype(jnp.int32)

    mesh = plsc.VectorSubcoreMesh(core_axis_name="c", subcore_axis_name="s")

    @jax.jit
    def run(table_arr, idx_arr):
        @pl.kernel(
            out_type=jax.ShapeDtypeStruct((n, dim), table_arr.dtype),
            mesh=mesh,
        )
        def gather_kernel(table_hbm, idx_hbm, out_hbm):
            def body(i_vmem, o_vmem):
                pltpu.sync_copy(table_hbm.at[i_vmem.at[0]], o_vmem)

            pltpu.emit_pipeline(
                body,
                grid=(n // _WINDOW,),
                in_specs=[pl.BlockSpec((1, _WINDOW), lambda i: (0, i))],
                out_specs=[pl.BlockSpec((_WINDOW, dim), lambda i: (i, 0))],
                core_axis_name=("c", "s"),
                dimension_semantics=(pltpu.PARALLEL,),
            )(idx_hbm, out_hbm)

        return gather_kernel(table_arr, idx_arr)

    out = run(table, idx)
    return out.reshape(batch, seq, dim)


# speedup vs baseline: 1.2330x; 1.2330x over previous
import jax
import jax.numpy as jnp
from jax.experimental import pallas as pl
from jax.experimental.pallas import tpu as pltpu
from jax.experimental.pallas import tpu_sc as plsc

_WINDOW = 256


def kernel(txt_inds, table):
    batch, seq = txt_inds.shape
    vocab, dim = table.shape
    n = batch * seq
    idx = txt_inds.reshape(1, n).astype(jnp.int32)

    mesh = plsc.VectorSubcoreMesh(core_axis_name="c", subcore_axis_name="s")

    @jax.jit
    def run(table_arr, idx_arr):
        @pl.kernel(
            out_type=jax.ShapeDtypeStruct((n, dim), table_arr.dtype),
            mesh=mesh,
        )
        def gather_kernel(table_hbm, idx_hbm, out_hbm):
            def body(i_vmem, o_vmem):
                pltpu.sync_copy(table_hbm.at[i_vmem.at[0]], o_vmem)

            pltpu.emit_pipeline(
                body,
                grid=(n // _WINDOW,),
                in_specs=[pl.BlockSpec((1, _WINDOW), lambda i: (0, i))],
                out_specs=[pl.BlockSpec((_WINDOW, dim), lambda i: (i, 0))],
                core_axis_name=("c", "s"),
                dimension_semantics=(pltpu.PARALLEL,),
            )(idx_hbm, out_hbm)

        return gather_kernel(table_arr, idx_arr)

    out = run(table, idx)
    return out.reshape(batch, seq, dim)
